# baseline (speedup 1.0000x reference)
import functools

import numpy as np
import jax
import jax.numpy as jnp
from jax import lax
from jax.experimental import pallas as pl
from jax.experimental.pallas import tpu as pltpu
from jax.experimental.pallas import tpu_sc as plsc

S = 256
K = 16
RAD = 0.3
R2 = np.float32(RAD * RAD)
INV_R = np.float32(1.0) / np.float32(RAD)
EPS = np.float32(1e-5)


def _fps_body(sxyz_ref, out_ref):
    X = sxyz_ref[0]
    Y = sxyz_ref[1]
    Z = sxyz_ref[2]
    B, N = X.shape
    iota = lax.broadcasted_iota(jnp.int32, (B, N), 1)
    big = jnp.int32(N)

    def body(i, dists):
        m = jnp.max(dists, axis=1, keepdims=True)
        far = jnp.min(jnp.where(dists == m, iota, big), axis=1)
        out_ref[pl.ds(i, 1), :] = far[None, :]
        oh = iota == far[:, None]
        cx = jnp.sum(jnp.where(oh, X, 0.0), axis=1, keepdims=True)
        cy = jnp.sum(jnp.where(oh, Y, 0.0), axis=1, keepdims=True)
        cz = jnp.sum(jnp.where(oh, Z, 0.0), axis=1, keepdims=True)
        dx = X - cx
        dy = Y - cy
        dz = Z - cz
        d = dx * dx + dy * dy + dz * dz
        return jnp.minimum(dists, d)

    dists0 = jnp.full((B, N), 1e10, dtype=jnp.float32)
    lax.fori_loop(0, S, body, dists0, unroll=4)


def _bqg_body(xyzT_ref, ind_ref, feat_ref, w1_ref, b1_ref,
              nxyz_ref, idx_ref, g_ref, o_ref):
    b = pl.program_id(0)
    N = xyzT_ref.shape[2]
    P = xyzT_ref[0]
    X = P[0:1, :]
    Y = P[1:2, :]
    Z = P[2:3, :]
    ind = ind_ref[0]
    iota_n = lax.broadcasted_iota(jnp.int32, (S, N), 1)
    oh = iota_n == ind
    nx = jnp.sum(jnp.where(oh, X, 0.0), axis=1, keepdims=True)
    ny = jnp.sum(jnp.where(oh, Y, 0.0), axis=1, keepdims=True)
    nz = jnp.sum(jnp.where(oh, Z, 0.0), axis=1, keepdims=True)
    nxyz = jnp.concatenate([nx, ny, nz], axis=1)
    nxyz_ref[0] = nxyz
    dx = nx - X
    dy = ny - Y
    dz = nz - Z
    d2 = dx * dx + dy * dy + dz * dz
    big = jnp.int32(N)
    cur = jnp.where(d2 < R2, iota_n, big)
    first = jnp.min(cur, axis=1, keepdims=True)
    cols = [first]
    cur = jnp.where(cur == first, big, cur)
    for _k in range(K - 1):
        nk = jnp.min(cur, axis=1, keepdims=True)
        cols.append(jnp.where(nk < big, nk, first))
        cur = jnp.where(cur == nk, big, cur)
    idx_ref[0] = jnp.concatenate(cols, axis=1) + b * N
    wx = w1_ref[0:3, :]
    wf = w1_ref[3:, :]
    dn = (((0,), (0,)), ((), ()))
    gf = lax.dot_general(feat_ref[0], wf, dn, preferred_element_type=jnp.float32)
    gx = lax.dot_general(P, wx, dn, preferred_element_type=jnp.float32)
    g_ref[0] = gf + gx * INV_R + b1_ref[...]
    o = lax.dot_general(nxyz, wx, (((1,), (0,)), ((), ())),
                        preferred_element_type=jnp.float32)
    o_ref[0] = o * INV_R


def _make_sc_gather(n_rows, d, n_idx):
    info = plsc.get_sparse_core_info()
    nw = info.num_cores * info.num_subcores
    per_w = n_idx // nw
    ch = 128
    n_ch = per_w // ch
    mesh = plsc.VectorSubcoreMesh(core_axis_name="c", subcore_axis_name="s")

    @functools.partial(
        pl.kernel, mesh=mesh,
        out_type=jax.ShapeDtypeStruct((n_idx, d), jnp.float32),
        scratch_types=[
            pltpu.VMEM((n_ch, ch), jnp.int32),
            pltpu.VMEM((ch, d), jnp.float32),
            pltpu.VMEM((ch, d), jnp.float32),
            pltpu.SemaphoreType.DMA,
            pltpu.SemaphoreType.DMA,
        ],
    )
    def gather(tab_hbm, idx_hbm, out_hbm, idx_v, buf0, buf1, sem0, sem1):
        wid = lax.axis_index("s") * info.num_cores + lax.axis_index("c")
        base = wid * per_w
        pltpu.sync_copy(idx_hbm.at[wid], idx_v)
        bufs = (buf0, buf1)
        sems = (sem0, sem1)
        cps = [None, None]
        for c in range(n_ch):
            p = c % 2
            cps[p] = pltpu.async_copy(tab_hbm.at[idx_v.at[c]], bufs[p], sems[p])
            if c > 0:
                q = (c - 1) % 2
                cps[q].wait()
                pltpu.sync_copy(bufs[q], out_hbm.at[pl.ds(base + (c - 1) * ch, ch)])
        q = (n_ch - 1) % 2
        cps[q].wait()
        pltpu.sync_copy(bufs[q], out_hbm.at[pl.ds(base + (n_ch - 1) * ch, ch)])

    return gather


def _stats1_body(hg_ref, o_ref, st_ref, acc):
    i = pl.program_id(0)
    nb = pl.num_programs(0)
    h = hg_ref[...]
    blk = h.shape[0]
    x = (h.reshape(blk // K, K, 128) - o_ref[...][:, None, :]).reshape(blk, 128)
    s1 = jnp.sum(x, axis=0, keepdims=True)
    s2 = jnp.sum(x * x, axis=0, keepdims=True)

    @pl.when(i == 0)
    def _():
        acc[...] = jnp.zeros_like(acc)

    acc[0:1, :] += s1
    acc[1:2, :] += s2

    @pl.when(i == nb - 1)
    def _():
        st_ref[...] = acc[...]


def _bn(x, st, gb, cnt):
    mu = st[0:1, :] * np.float32(1.0 / cnt)
    var = st[1:2, :] * np.float32(1.0 / cnt) - mu * mu
    rstd = lax.rsqrt(var + EPS)
    return jnp.maximum((x - mu) * rstd * gb[0:1, :] + gb[1:2, :], 0.0)


def _layer_sub_body(cnt, hg_ref, o_ref, st_ref, w_ref, gb_ref, out_ref, st2_ref, acc):
    i = pl.program_id(0)
    nb = pl.num_programs(0)
    h = hg_ref[...]
    blk = h.shape[0]
    x = (h.reshape(blk // K, K, 128) - o_ref[...][:, None, :]).reshape(blk, 128)
    xn = _bn(x, st_ref, gb_ref, cnt)
    y = jnp.dot(xn, w_ref[...], preferred_element_type=jnp.float32) + gb_ref[2:3, :]
    out_ref[...] = y

    @pl.when(i == 0)
    def _():
        acc[...] = jnp.zeros_like(acc)

    acc[0:1, :] += jnp.sum(y, axis=0, keepdims=True)
    acc[1:2, :] += jnp.sum(y * y, axis=0, keepdims=True)

    @pl.when(i == nb - 1)
    def _():
        st2_ref[...] = acc[...]


def _layer_body(cnt, a_ref, st_ref, w_ref, gb_ref, out_ref, st2_ref, acc):
    i = pl.program_id(0)
    nb = pl.num_programs(0)
    xn = _bn(a_ref[...], st_ref, gb_ref, cnt)
    y = jnp.dot(xn, w_ref[...], preferred_element_type=jnp.float32) + gb_ref[2:3, :]
    out_ref[...] = y

    @pl.when(i == 0)
    def _():
        acc[...] = jnp.zeros_like(acc)

    acc[0:1, :] += jnp.sum(y, axis=0, keepdims=True)
    acc[1:2, :] += jnp.sum(y * y, axis=0, keepdims=True)

    @pl.when(i == nb - 1)
    def _():
        st2_ref[...] = acc[...]


def _final_body(cnt, a_ref, st_ref, gb_ref, out_ref):
    xn = _bn(a_ref[...], st_ref, gb_ref, cnt)
    blk = xn.shape[0]
    m = jnp.max(xn.reshape(blk // K, K, 128), axis=1)
    out_ref[0] = m.T


def kernel(xyz, features, seed_xyz, W1, b1, g1, be1, W2, b2, g2, be2,
           W3, b3, g3, be3):
    B, N, _ = xyz.shape
    C = features.shape[1]
    f32 = jnp.float32

    sxyz = jnp.transpose(seed_xyz, (2, 0, 1))
    sampT = pl.pallas_call(
        _fps_body,
        out_shape=jax.ShapeDtypeStruct((S, B), jnp.int32),
    )(sxyz)
    sample_inds = sampT.T

    xyzT = jnp.transpose(xyz, (0, 2, 1))
    ind3 = sample_inds[:, :, None]
    b1r = b1[None, :]
    new_xyz, idxf, G, O = pl.pallas_call(
        _bqg_body,
        grid=(B,),
        in_specs=[
            pl.BlockSpec((1, 3, N), lambda b: (b, 0, 0)),
            pl.BlockSpec((1, S, 1), lambda b: (b, 0, 0)),
            pl.BlockSpec((1, C, N), lambda b: (b, 0, 0)),
            pl.BlockSpec((C + 3, 128), lambda b: (0, 0)),
            pl.BlockSpec((1, 128), lambda b: (0, 0)),
        ],
        out_specs=[
            pl.BlockSpec((1, S, 3), lambda b: (b, 0, 0)),
            pl.BlockSpec((1, S, K), lambda b: (b, 0, 0)),
            pl.BlockSpec((1, N, 128), lambda b: (b, 0, 0)),
            pl.BlockSpec((1, S, 128), lambda b: (b, 0, 0)),
        ],
        out_shape=[
            jax.ShapeDtypeStruct((B, S, 3), f32),
            jax.ShapeDtypeStruct((B, S, K), jnp.int32),
            jax.ShapeDtypeStruct((B, N, 128), f32),
            jax.ShapeDtypeStruct((B, S, 128), f32),
        ],
    )(xyzT, ind3, features, W1, b1r)

    n_idx = B * S * K
    Gf = G.reshape(B * N, 128)
    info = plsc.get_sparse_core_info()
    nw = info.num_cores * info.num_subcores
    idx_grp = idxf.reshape(nw, (n_idx // nw) // 128, 128)
    H1g = _make_sc_gather(B * N, 128, n_idx)(Gf, idx_grp)

    cnt = n_idx
    BLK = 4096
    nb = n_idx // BLK
    Of = O.reshape(B * S, 128)
    stats_spec = pl.BlockSpec((8, 128), lambda i: (0, 0))
    row_spec = pl.BlockSpec((BLK, 128), lambda i: (i, 0))
    o_spec = pl.BlockSpec((BLK // K, 128), lambda i: (i, 0))
    w_spec = pl.BlockSpec((128, 128), lambda i: (0, 0))
    gb_spec = pl.BlockSpec((3, 128), lambda i: (0, 0))
    acc = pltpu.VMEM((8, 128), f32)

    st1 = pl.pallas_call(
        _stats1_body,
        grid=(nb,),
        in_specs=[row_spec, o_spec],
        out_specs=stats_spec,
        out_shape=jax.ShapeDtypeStruct((8, 128), f32),
        scratch_shapes=[acc],
    )(H1g, Of)

    gb2 = jnp.stack([g1, be1, b2])
    A2, st2 = pl.pallas_call(
        functools.partial(_layer_sub_body, cnt),
        grid=(nb,),
        in_specs=[row_spec, o_spec, stats_spec, w_spec, gb_spec],
        out_specs=[row_spec, stats_spec],
        out_shape=[jax.ShapeDtypeStruct((n_idx, 128), f32),
                   jax.ShapeDtypeStruct((8, 128), f32)],
        scratch_shapes=[acc],
    )(H1g, Of, st1, W2, gb2)

    gb3 = jnp.stack([g2, be2, b3])
    A3, st3 = pl.pallas_call(
        functools.partial(_layer_body, cnt),
        grid=(nb,),
        in_specs=[row_spec, stats_spec, w_spec, gb_spec],
        out_specs=[row_spec, stats_spec],
        out_shape=[jax.ShapeDtypeStruct((n_idx, 128), f32),
                   jax.ShapeDtypeStruct((8, 128), f32)],
        scratch_shapes=[acc],
    )(A2, st2, W3, gb3)

    gb4 = jnp.stack([g3, be3, jnp.zeros_like(b3)])
    new_features = pl.pallas_call(
        functools.partial(_final_body, cnt),
        grid=(nb,),
        in_specs=[row_spec, stats_spec, gb_spec],
        out_specs=pl.BlockSpec((1, 128, S), lambda i: (i, 0, 0)),
        out_shape=jax.ShapeDtypeStruct((B, 128, S), f32),
    )(A3, st3, gb4)

    return (new_xyz, new_features, sample_inds)


# speedup vs baseline: 1.1411x; 1.1411x over previous
import functools

import numpy as np
import jax
import jax.numpy as jnp
from jax import lax
from jax.experimental import pallas as pl
from jax.experimental.pallas import tpu as pltpu
from jax.experimental.pallas import tpu_sc as plsc

S = 256
K = 16
RAD = 0.3
R2 = np.float32(RAD * RAD)
INV_R = np.float32(1.0) / np.float32(RAD)
EPS = np.float32(1e-5)


def _fps_body(sxyz_ref, out_ref):
    X = sxyz_ref[0]
    Y = sxyz_ref[1]
    Z = sxyz_ref[2]
    B, N = X.shape
    iota = lax.broadcasted_iota(jnp.int32, (B, N), 1)
    big = jnp.int32(N)

    def body(i, dists):
        m = jnp.max(dists, axis=1, keepdims=True)
        far = jnp.min(jnp.where(dists == m, iota, big), axis=1)
        out_ref[pl.ds(i, 1), :] = far[None, :]
        oh = iota == far[:, None]
        cx = jnp.sum(jnp.where(oh, X, 0.0), axis=1, keepdims=True)
        cy = jnp.sum(jnp.where(oh, Y, 0.0), axis=1, keepdims=True)
        cz = jnp.sum(jnp.where(oh, Z, 0.0), axis=1, keepdims=True)
        dx = X - cx
        dy = Y - cy
        dz = Z - cz
        d = dx * dx + dy * dy + dz * dz
        return jnp.minimum(dists, d)

    dists0 = jnp.full((B, N), 1e10, dtype=jnp.float32)
    lax.fori_loop(0, S, body, dists0, unroll=4)


def _bqg_body(xyzT_ref, ind_ref, feat_ref, w1_ref, b1_ref,
              nxyz_ref, idx_ref, g_ref, o_ref):
    b = pl.program_id(0)
    N = xyzT_ref.shape[2]
    P = xyzT_ref[0]
    X = P[0:1, :]
    Y = P[1:2, :]
    Z = P[2:3, :]
    ind = ind_ref[0]
    iota_n = lax.broadcasted_iota(jnp.int32, (S, N), 1)
    oh = iota_n == ind
    nx = jnp.sum(jnp.where(oh, X, 0.0), axis=1, keepdims=True)
    ny = jnp.sum(jnp.where(oh, Y, 0.0), axis=1, keepdims=True)
    nz = jnp.sum(jnp.where(oh, Z, 0.0), axis=1, keepdims=True)
    nxyz = jnp.concatenate([nx, ny, nz], axis=1)
    nxyz_ref[0] = nxyz
    dx = nx - X
    dy = ny - Y
    dz = nz - Z
    d2 = dx * dx + dy * dy + dz * dz
    big = jnp.int32(N)
    cur = jnp.where(d2 < R2, iota_n, big)
    first = jnp.min(cur, axis=1, keepdims=True)
    cols = [first]
    cur = jnp.where(cur == first, big, cur)
    for _k in range(K - 1):
        nk = jnp.min(cur, axis=1, keepdims=True)
        cols.append(jnp.where(nk < big, nk, first))
        cur = jnp.where(cur == nk, big, cur)
    idx_ref[0] = jnp.concatenate(cols, axis=1) + b * N
    wx = w1_ref[0:3, :]
    wf = w1_ref[3:, :]
    dn = (((0,), (0,)), ((), ()))
    gf = lax.dot_general(feat_ref[0], wf, dn, preferred_element_type=jnp.float32)
    gx = lax.dot_general(P, wx, dn, preferred_element_type=jnp.float32)
    g_ref[0] = gf + gx * INV_R + b1_ref[...]
    o = lax.dot_general(nxyz, wx, (((1,), (0,)), ((), ())),
                        preferred_element_type=jnp.float32)
    o_ref[0] = o * INV_R


def _make_sc_gather(n_rows, d, n_idx):
    info = plsc.get_sparse_core_info()
    nw = info.num_cores * info.num_subcores
    per_w = n_idx // nw
    ch = 128
    n_ch = per_w // ch
    mesh = plsc.VectorSubcoreMesh(core_axis_name="c", subcore_axis_name="s")

    @functools.partial(
        pl.kernel, mesh=mesh,
        out_type=jax.ShapeDtypeStruct((n_idx, d), jnp.float32),
        scratch_types=[
            pltpu.VMEM((n_ch, ch), jnp.int32),
            pltpu.VMEM((ch, d), jnp.float32),
            pltpu.VMEM((ch, d), jnp.float32),
            pltpu.SemaphoreType.DMA,
            pltpu.SemaphoreType.DMA,
        ],
    )
    def gather(tab_hbm, idx_hbm, out_hbm, idx_v, buf0, buf1, sem0, sem1):
        wid = lax.axis_index("s") * info.num_cores + lax.axis_index("c")
        base = wid * per_w
        pltpu.sync_copy(idx_hbm.at[wid], idx_v)
        bufs = (buf0, buf1)
        sems = (sem0, sem1)
        cps = [None, None]
        for c in range(n_ch):
            p = c % 2
            cps[p] = pltpu.async_copy(tab_hbm.at[idx_v.at[c]], bufs[p], sems[p])
            if c > 0:
                q = (c - 1) % 2
                cps[q].wait()
                pltpu.sync_copy(bufs[q], out_hbm.at[pl.ds(base + (c - 1) * ch, ch)])
        q = (n_ch - 1) % 2
        cps[q].wait()
        pltpu.sync_copy(bufs[q], out_hbm.at[pl.ds(base + (n_ch - 1) * ch, ch)])

    return gather


def _bn_v(x, s1, s2, g, be, cnt):
    mu = s1 * np.float32(1.0 / cnt)
    var = s2 * np.float32(1.0 / cnt) - mu * mu
    rstd = lax.rsqrt(var + EPS)
    return jnp.maximum((x - mu) * rstd * g + be, 0.0)


def _mlp_body(nb, blk, hg_ref, o_ref, w2_ref, w3_ref, gb2_ref, gb3_ref,
              gb4_ref, nf_ref, h1v, a2v, sems):
    cnt = np.float32(nb * blk)
    cps = []
    for i in range(nb):
        cp = pltpu.make_async_copy(hg_ref.at[pl.ds(i * blk, blk), :],
                                   h1v.at[pl.ds(i * blk, blk), :], sems.at[i])
        cp.start()
        cps.append(cp)
    s1 = jnp.zeros((1, 128), jnp.float32)
    s2 = jnp.zeros((1, 128), jnp.float32)
    for i in range(nb):
        cps[i].wait()
        h = h1v[pl.ds(i * blk, blk), :]
        o = o_ref[pl.ds(i * (blk // K), blk // K), :]
        x = (h.reshape(blk // K, K, 128) - o[:, None, :]).reshape(blk, 128)
        h1v[pl.ds(i * blk, blk), :] = x
        s1 = s1 + jnp.sum(x, axis=0, keepdims=True)
        s2 = s2 + jnp.sum(x * x, axis=0, keepdims=True)
    g, be, bias = gb2_ref[0:1, :], gb2_ref[1:2, :], gb2_ref[2:3, :]
    w2 = w2_ref[...]
    t1 = jnp.zeros((1, 128), jnp.float32)
    t2 = jnp.zeros((1, 128), jnp.float32)
    for i in range(nb):
        xn = _bn_v(h1v[pl.ds(i * blk, blk), :], s1, s2, g, be, cnt)
        y = jnp.dot(xn, w2, preferred_element_type=jnp.float32) + bias
        a2v[pl.ds(i * blk, blk), :] = y
        t1 = t1 + jnp.sum(y, axis=0, keepdims=True)
        t2 = t2 + jnp.sum(y * y, axis=0, keepdims=True)
    g, be, bias = gb3_ref[0:1, :], gb3_ref[1:2, :], gb3_ref[2:3, :]
    w3 = w3_ref[...]
    u1 = jnp.zeros((1, 128), jnp.float32)
    u2 = jnp.zeros((1, 128), jnp.float32)
    for i in range(nb):
        xn = _bn_v(a2v[pl.ds(i * blk, blk), :], t1, t2, g, be, cnt)
        y = jnp.dot(xn, w3, preferred_element_type=jnp.float32) + bias
        h1v[pl.ds(i * blk, blk), :] = y
        u1 = u1 + jnp.sum(y, axis=0, keepdims=True)
        u2 = u2 + jnp.sum(y * y, axis=0, keepdims=True)
    g, be = gb4_ref[0:1, :], gb4_ref[1:2, :]
    for i in range(nb):
        xn = _bn_v(h1v[pl.ds(i * blk, blk), :], u1, u2, g, be, cnt)
        m = jnp.max(xn.reshape(blk // K, K, 128), axis=1)
        nf_ref[i] = m.T


def kernel(xyz, features, seed_xyz, W1, b1, g1, be1, W2, b2, g2, be2,
           W3, b3, g3, be3):
    B, N, _ = xyz.shape
    C = features.shape[1]
    f32 = jnp.float32

    sxyz = jnp.transpose(seed_xyz, (2, 0, 1))
    sampT = pl.pallas_call(
        _fps_body,
        out_shape=jax.ShapeDtypeStruct((S, B), jnp.int32),
    )(sxyz)
    sample_inds = sampT.T

    xyzT = jnp.transpose(xyz, (0, 2, 1))
    ind3 = sample_inds[:, :, None]
    b1r = b1[None, :]
    new_xyz, idxf, G, O = pl.pallas_call(
        _bqg_body,
        grid=(B,),
        in_specs=[
            pl.BlockSpec((1, 3, N), lambda b: (b, 0, 0)),
            pl.BlockSpec((1, S, 1), lambda b: (b, 0, 0)),
            pl.BlockSpec((1, C, N), lambda b: (b, 0, 0)),
            pl.BlockSpec((C + 3, 128), lambda b: (0, 0)),
            pl.BlockSpec((1, 128), lambda b: (0, 0)),
        ],
        out_specs=[
            pl.BlockSpec((1, S, 3), lambda b: (b, 0, 0)),
            pl.BlockSpec((1, S, K), lambda b: (b, 0, 0)),
            pl.BlockSpec((1, N, 128), lambda b: (b, 0, 0)),
            pl.BlockSpec((1, S, 128), lambda b: (b, 0, 0)),
        ],
        out_shape=[
            jax.ShapeDtypeStruct((B, S, 3), f32),
            jax.ShapeDtypeStruct((B, S, K), jnp.int32),
            jax.ShapeDtypeStruct((B, N, 128), f32),
            jax.ShapeDtypeStruct((B, S, 128), f32),
        ],
    )(xyzT, ind3, features, W1, b1r)

    n_idx = B * S * K
    Gf = G.reshape(B * N, 128)
    info = plsc.get_sparse_core_info()
    nw = info.num_cores * info.num_subcores
    idx_grp = idxf.reshape(nw, (n_idx // nw) // 128, 128)
    H1g = _make_sc_gather(B * N, 128, n_idx)(Gf, idx_grp)

    BLK = 4096
    nb = n_idx // BLK
    Of = O.reshape(B * S, 128)
    gb2 = jnp.stack([g1, be1, b2])
    gb3 = jnp.stack([g2, be2, b3])
    gb4 = jnp.stack([g3, be3, jnp.zeros_like(b3)])
    new_features = pl.pallas_call(
        functools.partial(_mlp_body, nb, BLK),
        in_specs=[
            pl.BlockSpec(memory_space=pltpu.MemorySpace.HBM),
            pl.BlockSpec(memory_space=pltpu.VMEM),
            pl.BlockSpec(memory_space=pltpu.VMEM),
            pl.BlockSpec(memory_space=pltpu.VMEM),
            pl.BlockSpec(memory_space=pltpu.VMEM),
            pl.BlockSpec(memory_space=pltpu.VMEM),
            pl.BlockSpec(memory_space=pltpu.VMEM),
        ],
        out_specs=pl.BlockSpec(memory_space=pltpu.VMEM),
        out_shape=jax.ShapeDtypeStruct((B, 128, S), f32),
        scratch_shapes=[
            pltpu.VMEM((n_idx, 128), f32),
            pltpu.VMEM((n_idx, 128), f32),
            pltpu.SemaphoreType.DMA((nb,)),
        ],
    )(H1g, Of, W2, W3, gb2, gb3, gb4)

    return (new_xyz, new_features, sample_inds)
